# initial kernel scaffold (unmeasured)
import jax
import jax.numpy as jnp
from jax import lax
from jax.experimental import pallas as pl
from jax.experimental.pallas import tpu as pltpu


def kernel(
    x,
):
    def body(*refs):
        pass

    out_shape = jax.ShapeDtypeStruct(..., jnp.float32)
    return pl.pallas_call(body, out_shape=out_shape)(...)


# baseline (device time: 17188 ns/iter reference)
import jax
import jax.numpy as jnp
from jax import lax
from jax.experimental import pallas as pl
from jax.experimental.pallas import tpu as pltpu

N_DEV = 16


def kernel(x):
    m, n = x.shape

    def body(x_ref, out_ref, mine_ref, rbuf_ref, send_sems, recv_sems):
        my = lax.axis_index("i")

        mine_ref[...] = jnp.sum(x_ref[...], axis=0, keepdims=True)

        rdmas = []
        for f in range(1, N_DEV):
            tgt = (my + f) % N_DEV
            rdma = pltpu.make_async_remote_copy(
                src_ref=mine_ref,
                dst_ref=rbuf_ref.at[f - 1],
                send_sem=send_sems.at[f - 1],
                recv_sem=recv_sems.at[f - 1],
                device_id=(tgt,),
                device_id_type=pl.DeviceIdType.MESH,
            )
            rdma.start()
            rdmas.append(rdma)

        for rdma in rdmas:
            rdma.wait_recv()
        out_ref[...] = mine_ref[...] + jnp.sum(rbuf_ref[...], axis=0)

        for rdma in rdmas:
            rdma.wait_send()

    return pl.pallas_call(
        body,
        out_shape=jax.ShapeDtypeStruct((1, n), jnp.float32),
        in_specs=[pl.BlockSpec(memory_space=pltpu.VMEM)],
        out_specs=pl.BlockSpec(memory_space=pltpu.VMEM),
        scratch_shapes=[
            pltpu.VMEM((1, n), jnp.float32),
            pltpu.VMEM((N_DEV - 1, 1, n), jnp.float32),
            pltpu.SemaphoreType.DMA((N_DEV - 1,)),
            pltpu.SemaphoreType.DMA((N_DEV - 1,)),
        ],
    )(x)
